# baseline (speedup 1.0000x reference)
import jax
import jax.numpy as jnp
from jax import lax
from jax.experimental import pallas as pl
from jax.experimental.pallas import tpu as pltpu

B = 16
N = 16
D = 128
SEQ = 2048
GRP = 16
NG = SEQ // GRP
G = 64
NSPLIT = 4
SSPLIT = SEQ // NSPLIT


def _copy_kernel(k_in, v_in, kw_in, k_out, v_out, kw_out):
    k_out[...] = k_in[...]
    v_out[...] = v_in[...]
    kw_out[...] = kw_in[...]


_COPY_SPEC = dict(
    grid=(G,),
    in_specs=[
        pl.BlockSpec((1, NSPLIT, NG, GRP, D), lambda i: (i // NSPLIT, i % NSPLIT, 0, 0, 0)),
        pl.BlockSpec((1, NSPLIT, NG, GRP, D), lambda i: (i // NSPLIT, i % NSPLIT, 0, 0, 0)),
        pl.BlockSpec((1, SSPLIT, 2, N, N), lambda i: (i // NSPLIT, i % NSPLIT, 0, 0, 0)),
    ],
    out_specs=[
        pl.BlockSpec((1, NSPLIT, NG, GRP, D), lambda i: (i // NSPLIT, i % NSPLIT, 0, 0, 0)),
        pl.BlockSpec((1, NSPLIT, NG, GRP, D), lambda i: (i // NSPLIT, i % NSPLIT, 0, 0, 0)),
        pl.BlockSpec((1, SSPLIT, 2, N, N), lambda i: (i // NSPLIT, i % NSPLIT, 0, 0, 0)),
    ],
)


def _blend_kernel(pos_ref, k_val, v_val, kw_val, k_in, v_in, kw_in,
                  k_out, v_out, kw_out):
    sub = lax.broadcasted_iota(jnp.int32, (1, 1, 1, GRP, 1), 3)
    hit = sub == pos_ref[0] % GRP
    k_out[...] = jnp.where(hit, k_val[...], k_in[...])
    v_out[...] = jnp.where(hit, v_val[...], v_in[...])
    kw_out[...] = kw_val[...]


_BLEND_SPEC = pltpu.PrefetchScalarGridSpec(
    num_scalar_prefetch=1,
    grid=(1,),
    in_specs=[
        pl.BlockSpec((B, N, 1, 1, D), lambda i, pos: (0, 0, 0, 0, 0)),
        pl.BlockSpec((B, N, 1, 1, D), lambda i, pos: (0, 0, 0, 0, 0)),
        pl.BlockSpec((B, 1, 2, N, N), lambda i, pos: (0, 0, 0, 0, 0)),
        pl.BlockSpec((B, N, 1, GRP, D), lambda i, pos: (0, 0, pos[0] // GRP, 0, 0)),
        pl.BlockSpec((B, N, 1, GRP, D), lambda i, pos: (0, 0, pos[0] // GRP, 0, 0)),
        pl.BlockSpec((B, 1, 2, N, N), lambda i, pos: (0, pos[0], 0, 0, 0)),
    ],
    out_specs=[
        pl.BlockSpec((B, N, 1, GRP, D), lambda i, pos: (0, 0, pos[0] // GRP, 0, 0)),
        pl.BlockSpec((B, N, 1, GRP, D), lambda i, pos: (0, 0, pos[0] // GRP, 0, 0)),
        pl.BlockSpec((B, 1, 2, N, N), lambda i, pos: (0, pos[0], 0, 0, 0)),
    ],
)


def kernel(input_pos, k_val, v_val, kw_val, k_cache, v_cache, kw_cache):
    pos = input_pos.astype(jnp.int32) % SEQ
    dt = k_cache.dtype
    bc = lambda x: lax.bitcast_convert_type(x, jnp.bfloat16)

    k_c, v_c, kw_c = pl.pallas_call(
        _copy_kernel,
        out_shape=[
            jax.ShapeDtypeStruct((B, N, NG, GRP, D), jnp.bfloat16),
            jax.ShapeDtypeStruct((B, N, NG, GRP, D), jnp.bfloat16),
            jax.ShapeDtypeStruct((B, SEQ, 2, N, N), jnp.bfloat16),
        ],
        **_COPY_SPEC,
    )(
        bc(k_cache).reshape(B, N, NG, GRP, D),
        bc(v_cache).reshape(B, N, NG, GRP, D),
        bc(kw_cache),
    )
    k_out, v_out, kw_out = pl.pallas_call(
        _blend_kernel,
        grid_spec=_BLEND_SPEC,
        out_shape=[
            jax.ShapeDtypeStruct((B, N, NG, GRP, D), jnp.bfloat16),
            jax.ShapeDtypeStruct((B, N, NG, GRP, D), jnp.bfloat16),
            jax.ShapeDtypeStruct((B, SEQ, 2, N, N), jnp.bfloat16),
        ],
        input_output_aliases={4: 0, 5: 1, 6: 2},
    )(
        pos,
        bc(k_val).reshape(B, N, 1, 1, D),
        bc(v_val).reshape(B, N, 1, 1, D),
        bc(kw_val),
        k_c, v_c, kw_c,
    )
    return (
        lax.bitcast_convert_type(k_out.reshape(B, N, SEQ, D), dt),
        lax.bitcast_convert_type(v_out.reshape(B, N, SEQ, D), dt),
        lax.bitcast_convert_type(kw_out, dt),
    )


# speedup vs baseline: 1.0023x; 1.0023x over previous
import jax
import jax.numpy as jnp
from jax import lax
from jax.experimental import pallas as pl
from jax.experimental.pallas import tpu as pltpu

B = 16
N = 16
D = 128
SEQ = 2048
GRP = 16
NG = SEQ // GRP
NCH = 8
KWS = 256
SLOTS = 6
DEPTH = 8


def _copy_kernel(k_in, v_in, kw_in, k_out, v_out, kw_out,
                 kv_buf, kw_buf, kv_in_sem, kv_out_sem, kw_in_sem, kw_out_sem):
    kv_jobs = []
    for b in range(B):
        for h in range(N // NCH):
            s = pl.ds(h * NCH, NCH)
            kv_jobs.append((k_in.at[b, s], k_out.at[b, s]))
            kv_jobs.append((v_in.at[b, s], v_out.at[b, s]))
    kw_jobs = []
    for b in range(B):
        for q in range(SEQ // KWS):
            s = pl.ds(q * KWS, KWS)
            kw_jobs.append((kw_in.at[b, s], kw_out.at[b, s]))

    jobs = []
    ikv = ikw = 0
    while ikv < len(kv_jobs) or ikw < len(kw_jobs):
        if ikv < len(kv_jobs):
            jobs.append((False, kv_jobs[ikv], ikv % SLOTS))
            ikv += 1
        for _ in range(2):
            if ikw < len(kw_jobs):
                jobs.append((True, kw_jobs[ikw], ikw % SLOTS))
                ikw += 1

    def bufs(is_kw, slot):
        if is_kw:
            return kw_buf.at[slot], kw_in_sem.at[slot], kw_out_sem.at[slot]
        return kv_buf.at[slot], kv_in_sem.at[slot], kv_out_sem.at[slot]

    pending_out = {}
    in_objs = {}

    def issue_in(j):
        is_kw, (src, _), slot = jobs[j]
        buf, in_sem, _ = bufs(is_kw, slot)
        prev = pending_out.pop((is_kw, slot), None)
        if prev is not None:
            prev.wait()
        o = pltpu.make_async_copy(src, buf, in_sem)
        o.start()
        in_objs[j] = o

    n = len(jobs)
    for j in range(min(DEPTH, n)):
        issue_in(j)
    for j in range(n):
        in_objs.pop(j).wait()
        is_kw, (_, dst), slot = jobs[j]
        buf, _, out_sem = bufs(is_kw, slot)
        o = pltpu.make_async_copy(buf, dst, out_sem)
        o.start()
        pending_out[(is_kw, slot)] = o
        if j + DEPTH < n:
            issue_in(j + DEPTH)
    for o in pending_out.values():
        o.wait()


_COPY_SPEC = dict(
    grid=(1,),
    in_specs=[pl.BlockSpec(memory_space=pl.ANY)] * 3,
    out_specs=[pl.BlockSpec(memory_space=pl.ANY)] * 3,
    scratch_shapes=[
        pltpu.VMEM((SLOTS, NCH, NG, GRP, D), jnp.bfloat16),
        pltpu.VMEM((SLOTS, KWS, 2, N, N), jnp.bfloat16),
        pltpu.SemaphoreType.DMA((SLOTS,)),
        pltpu.SemaphoreType.DMA((SLOTS,)),
        pltpu.SemaphoreType.DMA((SLOTS,)),
        pltpu.SemaphoreType.DMA((SLOTS,)),
    ],
)


def _blend_kernel(pos_ref, k_val, v_val, kw_val, k_in, v_in, kw_in,
                  k_out, v_out, kw_out):
    sub = lax.broadcasted_iota(jnp.int32, (1, 1, 1, GRP, 1), 3)
    hit = sub == pos_ref[0] % GRP
    k_out[...] = jnp.where(hit, k_val[...], k_in[...])
    v_out[...] = jnp.where(hit, v_val[...], v_in[...])
    kw_out[...] = kw_val[...]


_BLEND_SPEC = pltpu.PrefetchScalarGridSpec(
    num_scalar_prefetch=1,
    grid=(1,),
    in_specs=[
        pl.BlockSpec((B, N, 1, 1, D), lambda i, pos: (0, 0, 0, 0, 0)),
        pl.BlockSpec((B, N, 1, 1, D), lambda i, pos: (0, 0, 0, 0, 0)),
        pl.BlockSpec((B, 1, 2, N, N), lambda i, pos: (0, 0, 0, 0, 0)),
        pl.BlockSpec((B, N, 1, GRP, D), lambda i, pos: (0, 0, pos[0] // GRP, 0, 0)),
        pl.BlockSpec((B, N, 1, GRP, D), lambda i, pos: (0, 0, pos[0] // GRP, 0, 0)),
        pl.BlockSpec((B, 1, 2, N, N), lambda i, pos: (0, pos[0], 0, 0, 0)),
    ],
    out_specs=[
        pl.BlockSpec((B, N, 1, GRP, D), lambda i, pos: (0, 0, pos[0] // GRP, 0, 0)),
        pl.BlockSpec((B, N, 1, GRP, D), lambda i, pos: (0, 0, pos[0] // GRP, 0, 0)),
        pl.BlockSpec((B, 1, 2, N, N), lambda i, pos: (0, pos[0], 0, 0, 0)),
    ],
)


def kernel(input_pos, k_val, v_val, kw_val, k_cache, v_cache, kw_cache):
    pos = input_pos.astype(jnp.int32) % SEQ
    dt = k_cache.dtype
    bc = lambda x: lax.bitcast_convert_type(x, jnp.bfloat16)

    k_c, v_c, kw_c = pl.pallas_call(
        _copy_kernel,
        out_shape=[
            jax.ShapeDtypeStruct((B, N, NG, GRP, D), jnp.bfloat16),
            jax.ShapeDtypeStruct((B, N, NG, GRP, D), jnp.bfloat16),
            jax.ShapeDtypeStruct((B, SEQ, 2, N, N), jnp.bfloat16),
        ],
        **_COPY_SPEC,
    )(
        bc(k_cache).reshape(B, N, NG, GRP, D),
        bc(v_cache).reshape(B, N, NG, GRP, D),
        bc(kw_cache),
    )
    k_out, v_out, kw_out = pl.pallas_call(
        _blend_kernel,
        grid_spec=_BLEND_SPEC,
        out_shape=[
            jax.ShapeDtypeStruct((B, N, NG, GRP, D), jnp.bfloat16),
            jax.ShapeDtypeStruct((B, N, NG, GRP, D), jnp.bfloat16),
            jax.ShapeDtypeStruct((B, SEQ, 2, N, N), jnp.bfloat16),
        ],
        input_output_aliases={4: 0, 5: 1, 6: 2},
    )(
        pos,
        bc(k_val).reshape(B, N, 1, 1, D),
        bc(v_val).reshape(B, N, 1, 1, D),
        bc(kw_val),
        k_c, v_c, kw_c,
    )
    return (
        lax.bitcast_convert_type(k_out.reshape(B, N, SEQ, D), dt),
        lax.bitcast_convert_type(v_out.reshape(B, N, SEQ, D), dt),
        lax.bitcast_convert_type(kw_out, dt),
    )
